# baseline (speedup 1.0000x reference)
import functools

import jax
import jax.numpy as jnp
from jax.experimental import pallas as pl
from jax.experimental.pallas import tpu as pltpu
from jax.experimental.pallas import tpu_sc as plsc

B = 4
N1 = 1024
N2 = 4096
CIN = 256
C = 64

QT = 512
NW = 32
QPW = (B * N2) // NW
CH = 128
NCH = QPW // CH
L = 16


def _tc_body(p1t_ref, p2_ref, x1_ref, x2_ref,
             w_in_ref, b_in_ref, g_in_ref, be_in_ref,
             w_out_ref, b_out_ref, g_out_ref, be_out_ref,
             i0_ref, i1_ref, i2_ref, w0_ref, w1_ref, w2_ref,
             x1n_ref, x2n_ref):
    b = pl.program_id(0)
    t = pl.program_id(1)

    @pl.when((b == 0) & (t == 0))
    def _mlp():
        def bn_relu(h, g, be):
            m = jnp.mean(h, axis=0, keepdims=True)
            cen = h - m
            v = jnp.mean(cen * cen, axis=0, keepdims=True)
            return jnp.maximum(cen * jax.lax.rsqrt(v + 1e-5) * g + be, 0.0)

        h1 = jax.lax.dot_general(x1_ref[...], w_in_ref[...],
                                 (((1,), (1,)), ((), ())),
                                 preferred_element_type=jnp.float32)
        h1 = h1 + b_in_ref[...]
        x1n_ref[...] = bn_relu(h1, g_in_ref[...], be_in_ref[...])

        h2 = jax.lax.dot_general(x2_ref[...], w_out_ref[...],
                                 (((1,), (1,)), ((), ())),
                                 preferred_element_type=jnp.float32)
        h2 = h2 + b_out_ref[...]
        x2n_ref[...] = bn_relu(h2, g_out_ref[...], be_out_ref[...])

    p1x = p1t_ref[0, 0:1, :]
    p1y = p1t_ref[0, 1:2, :]
    p1z = p1t_ref[0, 2:3, :]
    p2t = p2_ref[0]
    dx = p2t[:, 0:1] - p1x
    dy = p2t[:, 1:2] - p1y
    dz = p2t[:, 2:3] - p1z
    d = dx * dx + dy * dy + dz * dz

    colid = jax.lax.broadcasted_iota(jnp.int32, (QT, N1), 1)
    idxs = []
    dists = []
    for _ in range(3):
        m = jnp.min(d, axis=1, keepdims=True)
        sel = jnp.where(d == m, colid, N1)
        i = jnp.min(sel, axis=1, keepdims=True)
        idxs.append(i)
        dists.append(m)
        d = jnp.where(colid == i, jnp.float32(jnp.inf), d)

    r = [1.0 / (jnp.sqrt(m) + 1e-8) for m in dists]
    norm = r[0] + r[1] + r[2]
    i0_ref[0] = idxs[0] + b * N1
    i1_ref[0] = idxs[1] + b * N1
    i2_ref[0] = idxs[2] + b * N1
    w0_ref[0] = r[0] / norm
    w1_ref[0] = r[1] / norm
    w2_ref[0] = r[2] / norm


def _sc_body(x1n_hbm, i0_hbm, i1_hbm, i2_hbm, wa0_hbm, wa1_hbm, wa2_hbm,
             x2n_hbm, y_hbm,
             idx0_v, idx1_v, idx2_v, r0, r1, r2,
             w0_v, w1_v, w2_v, x2_v, y_v, s0, s1, s2):
    wid = jax.lax.axis_index("s") * 2 + jax.lax.axis_index("c")
    base = wid * QPW
    for step in range(NCH):
        gb = base + step * CH
        pltpu.sync_copy(i0_hbm.at[pl.ds(gb, CH)], idx0_v)
        pltpu.sync_copy(i1_hbm.at[pl.ds(gb, CH)], idx1_v)
        pltpu.sync_copy(i2_hbm.at[pl.ds(gb, CH)], idx2_v)
        d0 = pltpu.async_copy(x1n_hbm.at[idx0_v], r0, s0)
        d1 = pltpu.async_copy(x1n_hbm.at[idx1_v], r1, s1)
        d2 = pltpu.async_copy(x1n_hbm.at[idx2_v], r2, s2)
        pltpu.sync_copy(wa0_hbm.at[pl.ds(gb, CH)], w0_v)
        pltpu.sync_copy(wa1_hbm.at[pl.ds(gb, CH)], w1_v)
        pltpu.sync_copy(wa2_hbm.at[pl.ds(gb, CH)], w2_v)
        pltpu.sync_copy(x2n_hbm.at[pl.ds(gb, CH)], x2_v)
        d0.wait()
        d1.wait()
        d2.wait()

        def gbody(g, carry):
            gq = g * L
            w0g = w0_v[pl.ds(gq, L)]
            w1g = w1_v[pl.ds(gq, L)]
            w2g = w2_v[pl.ds(gq, L)]
            for u in range(L):
                q = gq + u
                w0 = jnp.broadcast_to(w0g[u], (L,))
                w1 = jnp.broadcast_to(w1g[u], (L,))
                w2 = jnp.broadcast_to(w2g[u], (L,))
                for v in range(C // L):
                    sl = pl.ds(v * L, L)
                    acc = (w0 * r0[q, sl] + w1 * r1[q, sl]
                           + w2 * r2[q, sl] + x2_v[q, sl])
                    y_v[q, sl] = acc
            return carry

        jax.lax.fori_loop(0, CH // L, gbody, 0)
        pltpu.sync_copy(y_v, y_hbm.at[pl.ds(gb, CH)])


@functools.lru_cache(maxsize=1)
def _sc_call():
    mesh = plsc.VectorSubcoreMesh(core_axis_name="c", subcore_axis_name="s")
    return pl.kernel(
        _sc_body,
        out_type=jax.ShapeDtypeStruct((B * N2, C), jnp.float32),
        mesh=mesh,
        scratch_types=[
            pltpu.VMEM((CH,), jnp.int32),
            pltpu.VMEM((CH,), jnp.int32),
            pltpu.VMEM((CH,), jnp.int32),
            pltpu.VMEM((CH, C), jnp.float32),
            pltpu.VMEM((CH, C), jnp.float32),
            pltpu.VMEM((CH, C), jnp.float32),
            pltpu.VMEM((CH,), jnp.float32),
            pltpu.VMEM((CH,), jnp.float32),
            pltpu.VMEM((CH,), jnp.float32),
            pltpu.VMEM((CH, C), jnp.float32),
            pltpu.VMEM((CH, C), jnp.float32),
            pltpu.SemaphoreType.DMA,
            pltpu.SemaphoreType.DMA,
            pltpu.SemaphoreType.DMA,
        ],
        compiler_params=pltpu.CompilerParams(use_tc_tiling_on_sc=False),
    )


def kernel(x1, p1, x2, p2, W_in, b_in, g_in, be_in,
           W_out, b_out, g_out, be_out):
    p1t = jnp.pad(jnp.transpose(p1, (0, 2, 1)), ((0, 0), (0, 5), (0, 0)))
    p2p = jnp.pad(p2, ((0, 0), (0, 0), (0, 5)))
    _whole = lambda b, t: (0, 0)
    _vec = lambda b, t: (0,)
    i0, i1, i2, w0, w1, w2, x1n, x2n = pl.pallas_call(
        _tc_body,
        grid=(B, N2 // QT),
        in_specs=[
            pl.BlockSpec((1, 8, N1), lambda b, t: (b, 0, 0)),
            pl.BlockSpec((1, QT, 8), lambda b, t: (b, t, 0)),
            pl.BlockSpec((B * N1, CIN), _whole),
            pl.BlockSpec((B * N2, C), _whole),
            pl.BlockSpec((C, CIN), _whole),
            pl.BlockSpec((C,), _vec),
            pl.BlockSpec((C,), _vec),
            pl.BlockSpec((C,), _vec),
            pl.BlockSpec((C, C), _whole),
            pl.BlockSpec((C,), _vec),
            pl.BlockSpec((C,), _vec),
            pl.BlockSpec((C,), _vec),
        ],
        out_specs=(
            [pl.BlockSpec((1, QT, 1), lambda b, t: (b, t, 0))] * 6
            + [pl.BlockSpec((B * N1, C), _whole),
               pl.BlockSpec((B * N2, C), _whole)]
        ),
        out_shape=(
            [jax.ShapeDtypeStruct((B, N2, 1), jnp.int32)] * 3
            + [jax.ShapeDtypeStruct((B, N2, 1), jnp.float32)] * 3
            + [jax.ShapeDtypeStruct((B * N1, C), jnp.float32),
               jax.ShapeDtypeStruct((B * N2, C), jnp.float32)]
        ),
    )(p1t, p2p, x1.reshape(B * N1, CIN), x2.reshape(B * N2, C),
      W_in, b_in, g_in, be_in, W_out, b_out, g_out, be_out)

    y = _sc_call()(x1n,
                   i0.reshape(B * N2), i1.reshape(B * N2),
                   i2.reshape(B * N2),
                   w0.reshape(B * N2), w1.reshape(B * N2),
                   w2.reshape(B * N2), x2n)
    return (y.reshape(B, N2, C), p2)


# speedup vs baseline: 1.0322x; 1.0322x over previous
import functools

import jax
import jax.numpy as jnp
from jax.experimental import pallas as pl
from jax.experimental.pallas import tpu as pltpu
from jax.experimental.pallas import tpu_sc as plsc

B = 4
N1 = 1024
N2 = 4096
CIN = 256
C = 64

QT = 1024
NW = 32
QPW = (B * N2) // NW
CH = 128
NCH = QPW // CH
L = 16


def _tc_body(p1t_ref, p2_ref, x1_ref, x2_ref,
             w_in_ref, b_in_ref, g_in_ref, be_in_ref,
             w_out_ref, b_out_ref, g_out_ref, be_out_ref,
             i0_ref, i1_ref, i2_ref, w0_ref, w1_ref, w2_ref,
             x1n_ref, x2n_ref):
    b = pl.program_id(0)
    t = pl.program_id(1)

    @pl.when((b == 0) & (t == 0))
    def _mlp():
        def bn_relu(h, g, be):
            m = jnp.mean(h, axis=0, keepdims=True)
            msq = jnp.mean(h * h, axis=0, keepdims=True)
            v = msq - m * m
            scale = g * jax.lax.rsqrt(v + 1e-5)
            shift = be - m * scale
            return jnp.maximum(h * scale + shift, 0.0)

        h1 = jax.lax.dot_general(x1_ref[...], w_in_ref[...],
                                 (((1,), (1,)), ((), ())),
                                 preferred_element_type=jnp.float32)
        h1 = h1 + b_in_ref[...]
        x1n_ref[...] = bn_relu(h1, g_in_ref[...], be_in_ref[...])

        h2 = jax.lax.dot_general(x2_ref[...], w_out_ref[...],
                                 (((1,), (1,)), ((), ())),
                                 preferred_element_type=jnp.float32)
        h2 = h2 + b_out_ref[...]
        x2n_ref[...] = bn_relu(h2, g_out_ref[...], be_out_ref[...])

    p1x = p1t_ref[0, 0:1, :]
    p1y = p1t_ref[0, 1:2, :]
    p1z = p1t_ref[0, 2:3, :]
    p2t = p2_ref[0]
    dx = p2t[:, 0:1] - p1x
    dy = p2t[:, 1:2] - p1y
    dz = p2t[:, 2:3] - p1z
    d = dx * dx + dy * dy + dz * dz

    colid = jax.lax.broadcasted_iota(jnp.int32, (QT, N1), 1)
    idxs = []
    dists = []
    for _ in range(3):
        m = jnp.min(d, axis=1, keepdims=True)
        sel = jnp.where(d == m, colid, N1)
        i = jnp.min(sel, axis=1, keepdims=True)
        idxs.append(i)
        dists.append(m)
        d = jnp.where(colid == i, jnp.float32(jnp.inf), d)

    r = [1.0 / (jnp.sqrt(m) + 1e-8) for m in dists]
    norm = r[0] + r[1] + r[2]
    i0_ref[0] = idxs[0] + b * N1
    i1_ref[0] = idxs[1] + b * N1
    i2_ref[0] = idxs[2] + b * N1
    w0_ref[0] = r[0] / norm
    w1_ref[0] = r[1] / norm
    w2_ref[0] = r[2] / norm


def _sc_body(x1n_hbm, i0_hbm, i1_hbm, i2_hbm, wa0_hbm, wa1_hbm, wa2_hbm,
             x2n_hbm, y_hbm,
             idx0_v, idx1_v, idx2_v, r0, r1, r2,
             w0_v, w1_v, w2_v, x2_v, y_v, s0, s1, s2):
    wid = jax.lax.axis_index("s") * 2 + jax.lax.axis_index("c")
    base = wid * QPW
    for step in range(NCH):
        gb = base + step * CH
        pltpu.sync_copy(i0_hbm.at[pl.ds(gb, CH)], idx0_v)
        pltpu.sync_copy(i1_hbm.at[pl.ds(gb, CH)], idx1_v)
        pltpu.sync_copy(i2_hbm.at[pl.ds(gb, CH)], idx2_v)
        d0 = pltpu.async_copy(x1n_hbm.at[idx0_v], r0, s0)
        d1 = pltpu.async_copy(x1n_hbm.at[idx1_v], r1, s1)
        d2 = pltpu.async_copy(x1n_hbm.at[idx2_v], r2, s2)
        pltpu.sync_copy(wa0_hbm.at[pl.ds(gb, CH)], w0_v)
        pltpu.sync_copy(wa1_hbm.at[pl.ds(gb, CH)], w1_v)
        pltpu.sync_copy(wa2_hbm.at[pl.ds(gb, CH)], w2_v)
        pltpu.sync_copy(x2n_hbm.at[pl.ds(gb, CH)], x2_v)
        d0.wait()
        d1.wait()
        d2.wait()

        def gbody(g, carry):
            gq = g * L
            w0g = w0_v[pl.ds(gq, L)]
            w1g = w1_v[pl.ds(gq, L)]
            w2g = w2_v[pl.ds(gq, L)]
            for u in range(L):
                q = gq + u
                w0 = jnp.broadcast_to(w0g[u], (L,))
                w1 = jnp.broadcast_to(w1g[u], (L,))
                w2 = jnp.broadcast_to(w2g[u], (L,))
                for v in range(C // L):
                    sl = pl.ds(v * L, L)
                    acc = (w0 * r0[q, sl] + w1 * r1[q, sl]
                           + w2 * r2[q, sl] + x2_v[q, sl])
                    y_v[q, sl] = acc
            return carry

        jax.lax.fori_loop(0, CH // L, gbody, 0)
        pltpu.sync_copy(y_v, y_hbm.at[pl.ds(gb, CH)])


@functools.lru_cache(maxsize=1)
def _sc_call():
    mesh = plsc.VectorSubcoreMesh(core_axis_name="c", subcore_axis_name="s")
    return pl.kernel(
        _sc_body,
        out_type=jax.ShapeDtypeStruct((B * N2, C), jnp.float32),
        mesh=mesh,
        scratch_types=[
            pltpu.VMEM((CH,), jnp.int32),
            pltpu.VMEM((CH,), jnp.int32),
            pltpu.VMEM((CH,), jnp.int32),
            pltpu.VMEM((CH, C), jnp.float32),
            pltpu.VMEM((CH, C), jnp.float32),
            pltpu.VMEM((CH, C), jnp.float32),
            pltpu.VMEM((CH,), jnp.float32),
            pltpu.VMEM((CH,), jnp.float32),
            pltpu.VMEM((CH,), jnp.float32),
            pltpu.VMEM((CH, C), jnp.float32),
            pltpu.VMEM((CH, C), jnp.float32),
            pltpu.SemaphoreType.DMA,
            pltpu.SemaphoreType.DMA,
            pltpu.SemaphoreType.DMA,
        ],
        compiler_params=pltpu.CompilerParams(use_tc_tiling_on_sc=False),
    )


def kernel(x1, p1, x2, p2, W_in, b_in, g_in, be_in,
           W_out, b_out, g_out, be_out):
    p1t = jnp.pad(jnp.transpose(p1, (0, 2, 1)), ((0, 0), (0, 5), (0, 0)))
    p2p = jnp.pad(p2, ((0, 0), (0, 0), (0, 5)))
    _whole = lambda b, t: (0, 0)
    _vec = lambda b, t: (0,)
    i0, i1, i2, w0, w1, w2, x1n, x2n = pl.pallas_call(
        _tc_body,
        grid=(B, N2 // QT),
        in_specs=[
            pl.BlockSpec((1, 8, N1), lambda b, t: (b, 0, 0)),
            pl.BlockSpec((1, QT, 8), lambda b, t: (b, t, 0)),
            pl.BlockSpec((B * N1, CIN), _whole),
            pl.BlockSpec((B * N2, C), _whole),
            pl.BlockSpec((C, CIN), _whole),
            pl.BlockSpec((C,), _vec),
            pl.BlockSpec((C,), _vec),
            pl.BlockSpec((C,), _vec),
            pl.BlockSpec((C, C), _whole),
            pl.BlockSpec((C,), _vec),
            pl.BlockSpec((C,), _vec),
            pl.BlockSpec((C,), _vec),
        ],
        out_specs=(
            [pl.BlockSpec((1, QT, 1), lambda b, t: (b, t, 0))] * 6
            + [pl.BlockSpec((B * N1, C), _whole),
               pl.BlockSpec((B * N2, C), _whole)]
        ),
        out_shape=(
            [jax.ShapeDtypeStruct((B, N2, 1), jnp.int32)] * 3
            + [jax.ShapeDtypeStruct((B, N2, 1), jnp.float32)] * 3
            + [jax.ShapeDtypeStruct((B * N1, C), jnp.float32),
               jax.ShapeDtypeStruct((B * N2, C), jnp.float32)]
        ),
    )(p1t, p2p, x1.reshape(B * N1, CIN), x2.reshape(B * N2, C),
      W_in, b_in, g_in, be_in, W_out, b_out, g_out, be_out)

    y = _sc_call()(x1n,
                   i0.reshape(B * N2), i1.reshape(B * N2),
                   i2.reshape(B * N2),
                   w0.reshape(B * N2), w1.reshape(B * N2),
                   w2.reshape(B * N2), x2n)
    return (y.reshape(B, N2, C), p2)


# speedup vs baseline: 1.0422x; 1.0097x over previous
import functools

import jax
import jax.numpy as jnp
from jax.experimental import pallas as pl
from jax.experimental.pallas import tpu as pltpu
from jax.experimental.pallas import tpu_sc as plsc

B = 4
N1 = 1024
N2 = 4096
CIN = 256
C = 64

QT = 2048
NW = 32
QPW = (B * N2) // NW
CH = 128
NCH = QPW // CH
L = 16


def _tc_body(p1t_ref, p2_ref, x1_ref, x2_ref,
             w_in_ref, b_in_ref, g_in_ref, be_in_ref,
             w_out_ref, b_out_ref, g_out_ref, be_out_ref,
             i0_ref, i1_ref, i2_ref, w0_ref, w1_ref, w2_ref,
             x1n_ref, x2n_ref):
    b = pl.program_id(0)
    t = pl.program_id(1)

    @pl.when((b == 0) & (t == 0))
    def _mlp():
        def bn_relu(h, g, be):
            m = jnp.mean(h, axis=0, keepdims=True)
            msq = jnp.mean(h * h, axis=0, keepdims=True)
            v = msq - m * m
            scale = g * jax.lax.rsqrt(v + 1e-5)
            shift = be - m * scale
            return jnp.maximum(h * scale + shift, 0.0)

        h1 = jax.lax.dot_general(x1_ref[...], w_in_ref[...],
                                 (((1,), (1,)), ((), ())),
                                 preferred_element_type=jnp.float32)
        h1 = h1 + b_in_ref[...]
        x1n_ref[...] = bn_relu(h1, g_in_ref[...], be_in_ref[...])

        h2 = jax.lax.dot_general(x2_ref[...], w_out_ref[...],
                                 (((1,), (1,)), ((), ())),
                                 preferred_element_type=jnp.float32)
        h2 = h2 + b_out_ref[...]
        x2n_ref[...] = bn_relu(h2, g_out_ref[...], be_out_ref[...])

    p1x = p1t_ref[0, 0:1, :]
    p1y = p1t_ref[0, 1:2, :]
    p1z = p1t_ref[0, 2:3, :]
    p2t = p2_ref[0]
    dx = p2t[:, 0:1] - p1x
    dy = p2t[:, 1:2] - p1y
    dz = p2t[:, 2:3] - p1z
    d = dx * dx + dy * dy + dz * dz

    colid = jax.lax.broadcasted_iota(jnp.int32, (QT, N1), 1)
    idxs = []
    dists = []
    for _ in range(3):
        m = jnp.min(d, axis=1, keepdims=True)
        sel = jnp.where(d == m, colid, N1)
        i = jnp.min(sel, axis=1, keepdims=True)
        idxs.append(i)
        dists.append(m)
        d = jnp.where(colid == i, jnp.float32(jnp.inf), d)

    r = [1.0 / (jnp.sqrt(m) + 1e-8) for m in dists]
    norm = r[0] + r[1] + r[2]
    i0_ref[0] = idxs[0] + b * N1
    i1_ref[0] = idxs[1] + b * N1
    i2_ref[0] = idxs[2] + b * N1
    w0_ref[0] = r[0] / norm
    w1_ref[0] = r[1] / norm
    w2_ref[0] = r[2] / norm


def _sc_body(x1n_hbm, i0_hbm, i1_hbm, i2_hbm, wa0_hbm, wa1_hbm, wa2_hbm,
             x2n_hbm, y_hbm,
             idx0_v, idx1_v, idx2_v, r0, r1, r2,
             w0_v, w1_v, w2_v, x2_v, y_v, s0, s1, s2):
    wid = jax.lax.axis_index("s") * 2 + jax.lax.axis_index("c")
    base = wid * QPW
    for step in range(NCH):
        gb = base + step * CH
        pltpu.sync_copy(i0_hbm.at[pl.ds(gb, CH)], idx0_v)
        pltpu.sync_copy(i1_hbm.at[pl.ds(gb, CH)], idx1_v)
        pltpu.sync_copy(i2_hbm.at[pl.ds(gb, CH)], idx2_v)
        d0 = pltpu.async_copy(x1n_hbm.at[idx0_v], r0, s0)
        d1 = pltpu.async_copy(x1n_hbm.at[idx1_v], r1, s1)
        d2 = pltpu.async_copy(x1n_hbm.at[idx2_v], r2, s2)
        pltpu.sync_copy(wa0_hbm.at[pl.ds(gb, CH)], w0_v)
        pltpu.sync_copy(wa1_hbm.at[pl.ds(gb, CH)], w1_v)
        pltpu.sync_copy(wa2_hbm.at[pl.ds(gb, CH)], w2_v)
        pltpu.sync_copy(x2n_hbm.at[pl.ds(gb, CH)], x2_v)
        d0.wait()
        d1.wait()
        d2.wait()

        def gbody(g, carry):
            gq = g * L
            w0g = w0_v[pl.ds(gq, L)]
            w1g = w1_v[pl.ds(gq, L)]
            w2g = w2_v[pl.ds(gq, L)]
            for u in range(L):
                q = gq + u
                w0 = jnp.broadcast_to(w0g[u], (L,))
                w1 = jnp.broadcast_to(w1g[u], (L,))
                w2 = jnp.broadcast_to(w2g[u], (L,))
                for v in range(C // L):
                    sl = pl.ds(v * L, L)
                    acc = (w0 * r0[q, sl] + w1 * r1[q, sl]
                           + w2 * r2[q, sl] + x2_v[q, sl])
                    y_v[q, sl] = acc
            return carry

        jax.lax.fori_loop(0, CH // L, gbody, 0)
        pltpu.sync_copy(y_v, y_hbm.at[pl.ds(gb, CH)])


@functools.lru_cache(maxsize=1)
def _sc_call():
    mesh = plsc.VectorSubcoreMesh(core_axis_name="c", subcore_axis_name="s")
    return pl.kernel(
        _sc_body,
        out_type=jax.ShapeDtypeStruct((B * N2, C), jnp.float32),
        mesh=mesh,
        scratch_types=[
            pltpu.VMEM((CH,), jnp.int32),
            pltpu.VMEM((CH,), jnp.int32),
            pltpu.VMEM((CH,), jnp.int32),
            pltpu.VMEM((CH, C), jnp.float32),
            pltpu.VMEM((CH, C), jnp.float32),
            pltpu.VMEM((CH, C), jnp.float32),
            pltpu.VMEM((CH,), jnp.float32),
            pltpu.VMEM((CH,), jnp.float32),
            pltpu.VMEM((CH,), jnp.float32),
            pltpu.VMEM((CH, C), jnp.float32),
            pltpu.VMEM((CH, C), jnp.float32),
            pltpu.SemaphoreType.DMA,
            pltpu.SemaphoreType.DMA,
            pltpu.SemaphoreType.DMA,
        ],
        compiler_params=pltpu.CompilerParams(use_tc_tiling_on_sc=False),
    )


def kernel(x1, p1, x2, p2, W_in, b_in, g_in, be_in,
           W_out, b_out, g_out, be_out):
    p1t = jnp.pad(jnp.transpose(p1, (0, 2, 1)), ((0, 0), (0, 5), (0, 0)))
    p2p = jnp.pad(p2, ((0, 0), (0, 0), (0, 5)))
    _whole = lambda b, t: (0, 0)
    _vec = lambda b, t: (0,)
    i0, i1, i2, w0, w1, w2, x1n, x2n = pl.pallas_call(
        _tc_body,
        grid=(B, N2 // QT),
        in_specs=[
            pl.BlockSpec((1, 8, N1), lambda b, t: (b, 0, 0)),
            pl.BlockSpec((1, QT, 8), lambda b, t: (b, t, 0)),
            pl.BlockSpec((B * N1, CIN), _whole),
            pl.BlockSpec((B * N2, C), _whole),
            pl.BlockSpec((C, CIN), _whole),
            pl.BlockSpec((C,), _vec),
            pl.BlockSpec((C,), _vec),
            pl.BlockSpec((C,), _vec),
            pl.BlockSpec((C, C), _whole),
            pl.BlockSpec((C,), _vec),
            pl.BlockSpec((C,), _vec),
            pl.BlockSpec((C,), _vec),
        ],
        out_specs=(
            [pl.BlockSpec((1, QT, 1), lambda b, t: (b, t, 0))] * 6
            + [pl.BlockSpec((B * N1, C), _whole),
               pl.BlockSpec((B * N2, C), _whole)]
        ),
        out_shape=(
            [jax.ShapeDtypeStruct((B, N2, 1), jnp.int32)] * 3
            + [jax.ShapeDtypeStruct((B, N2, 1), jnp.float32)] * 3
            + [jax.ShapeDtypeStruct((B * N1, C), jnp.float32),
               jax.ShapeDtypeStruct((B * N2, C), jnp.float32)]
        ),
    )(p1t, p2p, x1.reshape(B * N1, CIN), x2.reshape(B * N2, C),
      W_in, b_in, g_in, be_in, W_out, b_out, g_out, be_out)

    y = _sc_call()(x1n,
                   i0.reshape(B * N2), i1.reshape(B * N2),
                   i2.reshape(B * N2),
                   w0.reshape(B * N2), w1.reshape(B * N2),
                   w2.reshape(B * N2), x2n)
    return (y.reshape(B, N2, C), p2)


# speedup vs baseline: 1.0588x; 1.0159x over previous
import functools

import jax
import jax.numpy as jnp
from jax.experimental import pallas as pl
from jax.experimental.pallas import tpu as pltpu
from jax.experimental.pallas import tpu_sc as plsc

B = 4
N1 = 1024
N2 = 4096
CIN = 256
C = 64

QT = 2048
NW = 32
QPW = (B * N2) // NW
CH = 128
NCH = QPW // CH
L = 16


def _tc_body(p1t_ref, p2_ref, x1_ref, x2_ref,
             w_in_ref, b_in_ref, g_in_ref, be_in_ref,
             w_out_ref, b_out_ref, g_out_ref, be_out_ref,
             i0_ref, i1_ref, i2_ref, w0_ref, w1_ref, w2_ref,
             x1n_ref, x2n_ref):
    b = pl.program_id(0)
    t = pl.program_id(1)

    @pl.when((b == 0) & (t == 0))
    def _mlp():
        def bn_relu(h, g, be):
            m = jnp.mean(h, axis=0, keepdims=True)
            msq = jnp.mean(h * h, axis=0, keepdims=True)
            v = msq - m * m
            scale = g * jax.lax.rsqrt(v + 1e-5)
            shift = be - m * scale
            return jnp.maximum(h * scale + shift, 0.0)

        h1 = jax.lax.dot_general(x1_ref[...], w_in_ref[...],
                                 (((1,), (1,)), ((), ())),
                                 preferred_element_type=jnp.float32)
        h1 = h1 + b_in_ref[...]
        x1n_ref[...] = bn_relu(h1, g_in_ref[...], be_in_ref[...])

        h2 = jax.lax.dot_general(x2_ref[...], w_out_ref[...],
                                 (((1,), (1,)), ((), ())),
                                 preferred_element_type=jnp.float32)
        h2 = h2 + b_out_ref[...]
        x2n_ref[...] = bn_relu(h2, g_out_ref[...], be_out_ref[...])

    p1x = p1t_ref[0, 0:1, :]
    p1y = p1t_ref[0, 1:2, :]
    p1z = p1t_ref[0, 2:3, :]
    p2t = p2_ref[0]
    dx = p2t[:, 0:1] - p1x
    dy = p2t[:, 1:2] - p1y
    dz = p2t[:, 2:3] - p1z
    d = dx * dx + dy * dy + dz * dz

    colid = jax.lax.broadcasted_iota(jnp.int32, (QT, N1), 1)
    idxs = []
    dists = []
    for _ in range(3):
        m = jnp.min(d, axis=1, keepdims=True)
        sel = jnp.where(d == m, colid, N1)
        i = jnp.min(sel, axis=1, keepdims=True)
        idxs.append(i)
        dists.append(m)
        d = jnp.where(colid == i, jnp.float32(jnp.inf), d)

    r = [1.0 / (jnp.sqrt(m) + 1e-8) for m in dists]
    norm = r[0] + r[1] + r[2]
    i0_ref[0] = idxs[0] + b * N1
    i1_ref[0] = idxs[1] + b * N1
    i2_ref[0] = idxs[2] + b * N1
    w0_ref[0] = r[0] / norm
    w1_ref[0] = r[1] / norm
    w2_ref[0] = r[2] / norm


def _sc_body(x1n_hbm, i0_hbm, i1_hbm, i2_hbm, wa0_hbm, wa1_hbm, wa2_hbm,
             x2n_hbm, y_hbm,
             idx0_v, idx1_v, idx2_v, r0, r1, r2,
             w0_v, w1_v, w2_v, x2_v, y_v,
             sg0, sg1, sy0, sy1):
    wid = jax.lax.axis_index("s") * 2 + jax.lax.axis_index("c")
    base = wid * QPW
    sg = (sg0, sg1)
    sy = (sy0, sy1)

    def load_and_fire(k, p):
        gb = base + k * CH
        pltpu.sync_copy(i0_hbm.at[pl.ds(gb, CH)], idx0_v.at[p])
        pltpu.sync_copy(i1_hbm.at[pl.ds(gb, CH)], idx1_v.at[p])
        pltpu.sync_copy(i2_hbm.at[pl.ds(gb, CH)], idx2_v.at[p])
        ds = [pltpu.async_copy(x1n_hbm.at[idx0_v.at[p]], r0.at[p], sg[p]),
              pltpu.async_copy(x1n_hbm.at[idx1_v.at[p]], r1.at[p], sg[p]),
              pltpu.async_copy(x1n_hbm.at[idx2_v.at[p]], r2.at[p], sg[p])]
        pltpu.sync_copy(wa0_hbm.at[pl.ds(gb, CH)], w0_v.at[p])
        pltpu.sync_copy(wa1_hbm.at[pl.ds(gb, CH)], w1_v.at[p])
        pltpu.sync_copy(wa2_hbm.at[pl.ds(gb, CH)], w2_v.at[p])
        pltpu.sync_copy(x2n_hbm.at[pl.ds(gb, CH)], x2_v.at[p])
        return ds

    pend = {0: load_and_fire(0, 0)}
    yout = {}
    for k in range(NCH):
        p = k % 2
        if k + 1 < NCH:
            pend[k + 1] = load_and_fire(k + 1, (k + 1) % 2)
        for dcopy in pend.pop(k):
            dcopy.wait()
        if k >= 2:
            yout.pop(k - 2).wait()

        def gbody(g, carry):
            gq = g * L
            w0g = w0_v[p, pl.ds(gq, L)]
            w1g = w1_v[p, pl.ds(gq, L)]
            w2g = w2_v[p, pl.ds(gq, L)]
            for u in range(L):
                q = gq + u
                w0 = jnp.broadcast_to(w0g[u], (L,))
                w1 = jnp.broadcast_to(w1g[u], (L,))
                w2 = jnp.broadcast_to(w2g[u], (L,))
                for v in range(C // L):
                    sl = pl.ds(v * L, L)
                    acc = (w0 * r0[p, q, sl] + w1 * r1[p, q, sl]
                           + w2 * r2[p, q, sl] + x2_v[p, q, sl])
                    y_v[p, q, sl] = acc
            return carry

        jax.lax.fori_loop(0, CH // L, gbody, 0)
        yout[k] = pltpu.async_copy(y_v.at[p],
                                   y_hbm.at[pl.ds(base + k * CH, CH)], sy[p])
    for k in sorted(yout):
        yout[k].wait()


@functools.lru_cache(maxsize=1)
def _sc_call():
    mesh = plsc.VectorSubcoreMesh(core_axis_name="c", subcore_axis_name="s")
    return pl.kernel(
        _sc_body,
        out_type=jax.ShapeDtypeStruct((B * N2, C), jnp.float32),
        mesh=mesh,
        scratch_types=[
            pltpu.VMEM((2, CH), jnp.int32),
            pltpu.VMEM((2, CH), jnp.int32),
            pltpu.VMEM((2, CH), jnp.int32),
            pltpu.VMEM((2, CH, C), jnp.float32),
            pltpu.VMEM((2, CH, C), jnp.float32),
            pltpu.VMEM((2, CH, C), jnp.float32),
            pltpu.VMEM((2, CH), jnp.float32),
            pltpu.VMEM((2, CH), jnp.float32),
            pltpu.VMEM((2, CH), jnp.float32),
            pltpu.VMEM((2, CH, C), jnp.float32),
            pltpu.VMEM((2, CH, C), jnp.float32),
            pltpu.SemaphoreType.DMA,
            pltpu.SemaphoreType.DMA,
            pltpu.SemaphoreType.DMA,
            pltpu.SemaphoreType.DMA,
        ],
        compiler_params=pltpu.CompilerParams(use_tc_tiling_on_sc=False),
    )


def kernel(x1, p1, x2, p2, W_in, b_in, g_in, be_in,
           W_out, b_out, g_out, be_out):
    p1t = jnp.pad(jnp.transpose(p1, (0, 2, 1)), ((0, 0), (0, 5), (0, 0)))
    p2p = jnp.pad(p2, ((0, 0), (0, 0), (0, 5)))
    _whole = lambda b, t: (0, 0)
    _vec = lambda b, t: (0,)
    i0, i1, i2, w0, w1, w2, x1n, x2n = pl.pallas_call(
        _tc_body,
        grid=(B, N2 // QT),
        in_specs=[
            pl.BlockSpec((1, 8, N1), lambda b, t: (b, 0, 0)),
            pl.BlockSpec((1, QT, 8), lambda b, t: (b, t, 0)),
            pl.BlockSpec((B * N1, CIN), _whole),
            pl.BlockSpec((B * N2, C), _whole),
            pl.BlockSpec((C, CIN), _whole),
            pl.BlockSpec((C,), _vec),
            pl.BlockSpec((C,), _vec),
            pl.BlockSpec((C,), _vec),
            pl.BlockSpec((C, C), _whole),
            pl.BlockSpec((C,), _vec),
            pl.BlockSpec((C,), _vec),
            pl.BlockSpec((C,), _vec),
        ],
        out_specs=(
            [pl.BlockSpec((1, QT, 1), lambda b, t: (b, t, 0))] * 6
            + [pl.BlockSpec((B * N1, C), _whole),
               pl.BlockSpec((B * N2, C), _whole)]
        ),
        out_shape=(
            [jax.ShapeDtypeStruct((B, N2, 1), jnp.int32)] * 3
            + [jax.ShapeDtypeStruct((B, N2, 1), jnp.float32)] * 3
            + [jax.ShapeDtypeStruct((B * N1, C), jnp.float32),
               jax.ShapeDtypeStruct((B * N2, C), jnp.float32)]
        ),
    )(p1t, p2p, x1.reshape(B * N1, CIN), x2.reshape(B * N2, C),
      W_in, b_in, g_in, be_in, W_out, b_out, g_out, be_out)

    y = _sc_call()(x1n,
                   i0.reshape(B * N2), i1.reshape(B * N2),
                   i2.reshape(B * N2),
                   w0.reshape(B * N2), w1.reshape(B * N2),
                   w2.reshape(B * N2), x2n)
    return (y.reshape(B, N2, C), p2)
